# baseline (speedup 1.0000x reference)
import jax
import jax.numpy as jnp
from jax.experimental import pallas as pl
from jax.experimental.pallas import tpu as pltpu

L = 15
SCALE_BOUND = 0.11
LIKELIHOOD_BOUND = 1e-09
_INV_SQRT2 = 0.7071067811865476

CHR = 512


def _compute(w2_ref, nbb_ref, x, s, m, o_ref, l_ref, slot):
    bx = x * w2_ref[L]
    acc = w2_ref[0] * jnp.tanh(bx + nbb_ref[0])
    for i in range(1, L):
        acc = acc + w2_ref[i] * jnp.tanh(bx + nbb_ref[i])
    o_ref[slot] = acc + m
    sb = jnp.maximum(s, SCALE_BOUND)
    rk = _INV_SQRT2 / sb
    zu = (0.5 - acc) * rk
    zl = (-0.5 - acc) * rk
    lik = 0.5 * (jax.lax.erf(zu) - jax.lax.erf(zl))
    l_ref[slot] = jnp.maximum(lik, LIKELIHOOD_BOUND)


def _body(w2_ref, nbb_ref, x_hbm, s_hbm, m_hbm, out_hbm, lik_hbm,
          xb, sb_, mb, ob, lb, sin, sout):
    R = x_hbm.shape[0]
    nch = R // CHR

    def in_copies(k):
        slot = k % 2
        rows = pl.ds(k * CHR, CHR)
        return [
            pltpu.make_async_copy(x_hbm.at[rows], xb.at[slot], sin.at[slot, 0]),
            pltpu.make_async_copy(s_hbm.at[rows], sb_.at[slot], sin.at[slot, 1]),
            pltpu.make_async_copy(m_hbm.at[rows], mb.at[slot], sin.at[slot, 2]),
        ]

    def out_copies(k):
        slot = k % 2
        rows = pl.ds(k * CHR, CHR)
        return [
            pltpu.make_async_copy(ob.at[slot], out_hbm.at[rows], sout.at[slot, 0]),
            pltpu.make_async_copy(lb.at[slot], lik_hbm.at[rows], sout.at[slot, 1]),
        ]

    for c in in_copies(0):
        c.start()
    for k in range(nch):
        if k + 1 < nch:
            for c in in_copies(k + 1):
                c.start()
        for c in in_copies(k):
            c.wait()
        if k >= 2:
            for c in out_copies(k - 2):
                c.wait()
        slot = k % 2
        _compute(w2_ref, nbb_ref, xb[slot], sb_[slot], mb[slot], ob, lb, slot)
        for c in out_copies(k):
            c.start()
    for c in out_copies(nch - 2):
        c.wait()
    for c in out_copies(nch - 1):
        c.wait()


def kernel(inputs, scales, means, w, b, beta):
    B, C, H, W = inputs.shape
    R = B * H * W

    x2 = jnp.transpose(inputs, (0, 2, 3, 1)).reshape(R, C)
    s2 = jnp.transpose(scales, (0, 2, 3, 1)).reshape(R, C)
    m2 = jnp.transpose(means, (0, 2, 3, 1)).reshape(R, C)

    w2 = jnp.concatenate([w * 0.5, beta.reshape(1)]).astype(jnp.float32)
    nbb = (-beta * b).astype(jnp.float32)

    out2, lik2 = pl.pallas_call(
        _body,
        in_specs=[
            pl.BlockSpec(memory_space=pltpu.SMEM),
            pl.BlockSpec(memory_space=pltpu.SMEM),
            pl.BlockSpec(memory_space=pl.ANY),
            pl.BlockSpec(memory_space=pl.ANY),
            pl.BlockSpec(memory_space=pl.ANY),
        ],
        out_specs=[
            pl.BlockSpec(memory_space=pl.ANY),
            pl.BlockSpec(memory_space=pl.ANY),
        ],
        out_shape=[
            jax.ShapeDtypeStruct((R, C), jnp.float32),
            jax.ShapeDtypeStruct((R, C), jnp.float32),
        ],
        scratch_shapes=[
            pltpu.VMEM((2, CHR, C), jnp.float32),
            pltpu.VMEM((2, CHR, C), jnp.float32),
            pltpu.VMEM((2, CHR, C), jnp.float32),
            pltpu.VMEM((2, CHR, C), jnp.float32),
            pltpu.VMEM((2, CHR, C), jnp.float32),
            pltpu.SemaphoreType.DMA((2, 3)),
            pltpu.SemaphoreType.DMA((2, 2)),
        ],
    )(w2, nbb, x2, s2, m2)
    out = jnp.transpose(out2.reshape(B, H, W, C), (0, 3, 1, 2))
    lik = jnp.transpose(lik2.reshape(B, H, W, C), (0, 3, 1, 2))
    return out, lik


# speedup vs baseline: 1.0257x; 1.0257x over previous
import jax
import jax.numpy as jnp
from jax.experimental import pallas as pl
from jax.experimental.pallas import tpu as pltpu

L = 15
SCALE_BOUND = 0.11
LIKELIHOOD_BOUND = 1e-09
_INV_SQRT2 = 0.7071067811865476

CHR = 512


def _compute(w2_ref, nbb_ref, x, s, m, o_ref, l_ref, slot):
    bx = x * w2_ref[L]
    acc = w2_ref[0] * jnp.tanh(bx + nbb_ref[0])
    for i in range(1, L):
        acc = acc + w2_ref[i] * jnp.tanh(bx + nbb_ref[i])
    o_ref[slot] = acc + m
    sb = jnp.maximum(s, SCALE_BOUND)
    rk = _INV_SQRT2 / sb
    zu = (0.5 - acc) * rk
    zl = (-0.5 - acc) * rk
    lik = 0.5 * (jax.lax.erf(zu) - jax.lax.erf(zl))
    l_ref[slot] = jnp.maximum(lik, LIKELIHOOD_BOUND)


def _body(w2_ref, nbb_ref, x_hbm, s_hbm, m_hbm, out_hbm, lik_hbm,
          xb, sb_, mb, ob, lb, sin, sout):
    R = x_hbm.shape[0]
    nch = R // CHR

    def in_copies(k):
        slot = k % 3
        rows = pl.ds(k * CHR, CHR)
        return [
            pltpu.make_async_copy(x_hbm.at[rows], xb.at[slot], sin.at[slot, 0]),
            pltpu.make_async_copy(s_hbm.at[rows], sb_.at[slot], sin.at[slot, 1]),
            pltpu.make_async_copy(m_hbm.at[rows], mb.at[slot], sin.at[slot, 2]),
        ]

    def out_copies(k):
        slot = k % 3
        rows = pl.ds(k * CHR, CHR)
        return [
            pltpu.make_async_copy(ob.at[slot], out_hbm.at[rows], sout.at[slot, 0]),
            pltpu.make_async_copy(lb.at[slot], lik_hbm.at[rows], sout.at[slot, 1]),
        ]

    for c in in_copies(0):
        c.start()
    for c in in_copies(1):
        c.start()
    for k in range(nch):
        if k + 2 < nch:
            for c in in_copies(k + 2):
                c.start()
        for c in in_copies(k):
            c.wait()
        if k >= 3:
            for c in out_copies(k - 3):
                c.wait()
        slot = k % 3
        _compute(w2_ref, nbb_ref, xb[slot], sb_[slot], mb[slot], ob, lb, slot)
        for c in out_copies(k):
            c.start()
    for c in out_copies(nch - 3):
        c.wait()
    for c in out_copies(nch - 2):
        c.wait()
    for c in out_copies(nch - 1):
        c.wait()


def kernel(inputs, scales, means, w, b, beta):
    B, C, H, W = inputs.shape
    R = B * H * W

    x2 = jnp.transpose(inputs, (0, 2, 3, 1)).reshape(R, C)
    s2 = jnp.transpose(scales, (0, 2, 3, 1)).reshape(R, C)
    m2 = jnp.transpose(means, (0, 2, 3, 1)).reshape(R, C)

    w2 = jnp.concatenate([w * 0.5, beta.reshape(1)]).astype(jnp.float32)
    nbb = (-beta * b).astype(jnp.float32)

    out2, lik2 = pl.pallas_call(
        _body,
        in_specs=[
            pl.BlockSpec(memory_space=pltpu.SMEM),
            pl.BlockSpec(memory_space=pltpu.SMEM),
            pl.BlockSpec(memory_space=pl.ANY),
            pl.BlockSpec(memory_space=pl.ANY),
            pl.BlockSpec(memory_space=pl.ANY),
        ],
        out_specs=[
            pl.BlockSpec(memory_space=pl.ANY),
            pl.BlockSpec(memory_space=pl.ANY),
        ],
        out_shape=[
            jax.ShapeDtypeStruct((R, C), jnp.float32),
            jax.ShapeDtypeStruct((R, C), jnp.float32),
        ],
        scratch_shapes=[
            pltpu.VMEM((3, CHR, C), jnp.float32),
            pltpu.VMEM((3, CHR, C), jnp.float32),
            pltpu.VMEM((3, CHR, C), jnp.float32),
            pltpu.VMEM((3, CHR, C), jnp.float32),
            pltpu.VMEM((3, CHR, C), jnp.float32),
            pltpu.SemaphoreType.DMA((3, 3)),
            pltpu.SemaphoreType.DMA((3, 2)),
        ],
    )(w2, nbb, x2, s2, m2)
    out = jnp.transpose(out2.reshape(B, H, W, C), (0, 3, 1, 2))
    lik = jnp.transpose(lik2.reshape(B, H, W, C), (0, 3, 1, 2))
    return out, lik


# speedup vs baseline: 1.0468x; 1.0206x over previous
import jax
import jax.numpy as jnp
from jax.experimental import pallas as pl
from jax.experimental.pallas import tpu as pltpu

L = 15
SCALE_BOUND = 0.11
LIKELIHOOD_BOUND = 1e-09
_INV_SQRT2 = 0.7071067811865476

CHR = 256


def _compute(w2_ref, nbb_ref, x, s, m, o_ref, l_ref, slot):
    bx = x * w2_ref[L]
    acc = w2_ref[0] * jnp.tanh(bx + nbb_ref[0])
    for i in range(1, L):
        acc = acc + w2_ref[i] * jnp.tanh(bx + nbb_ref[i])
    o_ref[slot] = acc + m
    sb = jnp.maximum(s, SCALE_BOUND)
    rk = _INV_SQRT2 / sb
    zu = (0.5 - acc) * rk
    zl = (-0.5 - acc) * rk
    lik = 0.5 * (jax.lax.erf(zu) - jax.lax.erf(zl))
    l_ref[slot] = jnp.maximum(lik, LIKELIHOOD_BOUND)


def _body(w2_ref, nbb_ref, x_hbm, s_hbm, m_hbm, out_hbm, lik_hbm,
          xb, sb_, mb, ob, lb, sin, sout):
    R = x_hbm.shape[0]
    nch = R // CHR

    def in_copies(k):
        slot = k % 3
        rows = pl.ds(k * CHR, CHR)
        return [
            pltpu.make_async_copy(x_hbm.at[rows], xb.at[slot], sin.at[slot, 0]),
            pltpu.make_async_copy(s_hbm.at[rows], sb_.at[slot], sin.at[slot, 1]),
            pltpu.make_async_copy(m_hbm.at[rows], mb.at[slot], sin.at[slot, 2]),
        ]

    def out_copies(k):
        slot = k % 3
        rows = pl.ds(k * CHR, CHR)
        return [
            pltpu.make_async_copy(ob.at[slot], out_hbm.at[rows], sout.at[slot, 0]),
            pltpu.make_async_copy(lb.at[slot], lik_hbm.at[rows], sout.at[slot, 1]),
        ]

    for c in in_copies(0):
        c.start()
    for c in in_copies(1):
        c.start()
    for k in range(nch):
        if k + 2 < nch:
            for c in in_copies(k + 2):
                c.start()
        for c in in_copies(k):
            c.wait()
        if k >= 3:
            for c in out_copies(k - 3):
                c.wait()
        slot = k % 3
        _compute(w2_ref, nbb_ref, xb[slot], sb_[slot], mb[slot], ob, lb, slot)
        for c in out_copies(k):
            c.start()
    for c in out_copies(nch - 3):
        c.wait()
    for c in out_copies(nch - 2):
        c.wait()
    for c in out_copies(nch - 1):
        c.wait()


def kernel(inputs, scales, means, w, b, beta):
    B, C, H, W = inputs.shape
    R = B * H * W

    x2 = jnp.transpose(inputs, (0, 2, 3, 1)).reshape(R, C)
    s2 = jnp.transpose(scales, (0, 2, 3, 1)).reshape(R, C)
    m2 = jnp.transpose(means, (0, 2, 3, 1)).reshape(R, C)

    w2 = jnp.concatenate([w * 0.5, beta.reshape(1)]).astype(jnp.float32)
    nbb = (-beta * b).astype(jnp.float32)

    out2, lik2 = pl.pallas_call(
        _body,
        in_specs=[
            pl.BlockSpec(memory_space=pltpu.SMEM),
            pl.BlockSpec(memory_space=pltpu.SMEM),
            pl.BlockSpec(memory_space=pl.ANY),
            pl.BlockSpec(memory_space=pl.ANY),
            pl.BlockSpec(memory_space=pl.ANY),
        ],
        out_specs=[
            pl.BlockSpec(memory_space=pl.ANY),
            pl.BlockSpec(memory_space=pl.ANY),
        ],
        out_shape=[
            jax.ShapeDtypeStruct((R, C), jnp.float32),
            jax.ShapeDtypeStruct((R, C), jnp.float32),
        ],
        scratch_shapes=[
            pltpu.VMEM((3, CHR, C), jnp.float32),
            pltpu.VMEM((3, CHR, C), jnp.float32),
            pltpu.VMEM((3, CHR, C), jnp.float32),
            pltpu.VMEM((3, CHR, C), jnp.float32),
            pltpu.VMEM((3, CHR, C), jnp.float32),
            pltpu.SemaphoreType.DMA((3, 3)),
            pltpu.SemaphoreType.DMA((3, 2)),
        ],
    )(w2, nbb, x2, s2, m2)
    out = jnp.transpose(out2.reshape(B, H, W, C), (0, 3, 1, 2))
    lik = jnp.transpose(lik2.reshape(B, H, W, C), (0, 3, 1, 2))
    return out, lik
